# baseline (speedup 1.0000x reference)
import functools

import jax
import jax.numpy as jnp
import numpy as np
from jax import lax
from jax.experimental import pallas as pl
from jax.experimental.pallas import tpu as pltpu
from jax.experimental.pallas import tpu_sc as plsc

NC = 2
NS = 16
K = 40
KG = 8


def _mm_node_body(x_ref, w1_ref, wr_ref, y_ref, r_ref):
    x = x_ref[...]
    y_ref[...] = jnp.dot(x, w1_ref[...], preferred_element_type=jnp.float32)
    r_ref[...] = jnp.dot(x, wr_ref[...], preferred_element_type=jnp.float32)


def _mm_node(x, w1, wr, bn, half):
    n = x.shape[0]
    d = x.shape[1]
    nb = n // bn
    return pl.pallas_call(
        _mm_node_body,
        grid=(nb, NC),
        in_specs=[
            pl.BlockSpec((bn, d), lambda i, c: (i, 0)),
            pl.BlockSpec((d, half), lambda i, c: (0, c)),
            pl.BlockSpec((d, half), lambda i, c: (0, c)),
        ],
        out_specs=[
            pl.BlockSpec((bn, half), lambda i, c, _nb=nb: (c * _nb + i, 0)),
            pl.BlockSpec((bn, half), lambda i, c, _nb=nb: (c * _nb + i, 0)),
        ],
        out_shape=[
            jax.ShapeDtypeStruct((NC * n, half), jnp.float32),
            jax.ShapeDtypeStruct((NC * n, half), jnp.float32),
        ],
    )(x, w1, wr)


def _mm_edge_body(ea_ref, w2_ref, b_ref, e_ref):
    e_ref[...] = (jnp.dot(ea_ref[...], w2_ref[...],
                          preferred_element_type=jnp.float32)
                  + b_ref[...])


def _mm_edge(ea, w2, b2, be, half):
    e = ea.shape[0]
    de = ea.shape[1]
    eb = e // be
    return pl.pallas_call(
        _mm_edge_body,
        grid=(eb, NC),
        in_specs=[
            pl.BlockSpec((be, de), lambda i, c: (i, 0)),
            pl.BlockSpec((de, half), lambda i, c: (0, c)),
            pl.BlockSpec((1, half), lambda i, c: (0, c)),
        ],
        out_specs=pl.BlockSpec((be, half), lambda i, c, _eb=eb: (c * _eb + i, 0)),
        out_shape=jax.ShapeDtypeStruct((NC * e, half), jnp.float32),
    )(ea, w2, b2)


def _merge_body(a_ref, h_ref):
    h_ref[...] = jnp.maximum(a_ref[...], 0.0)


def _merge_relu(agg2, n, bn, half):
    nb = n // bn
    return pl.pallas_call(
        _merge_body,
        grid=(nb, NC),
        in_specs=[pl.BlockSpec((bn, half), lambda i, c, _nb=nb: (c * _nb + i, 0))],
        out_specs=pl.BlockSpec((bn, half), lambda i, c: (i, c)),
        out_shape=jax.ShapeDtypeStruct((n, NC * half), jnp.float32),
    )(agg2)


def _sc_edge_kernel(n, e, half, kchunks):
    eps = e // NS
    rows = (n // NS) // 16 * 16
    rem = n - NS * rows
    mesh = plsc.VectorSubcoreMesh(core_axis_name="c", subcore_axis_name="s")

    @functools.partial(
        pl.kernel,
        out_type=jax.ShapeDtypeStruct((NC * n, half), jnp.float32),
        mesh=mesh,
        scratch_types=[
            pltpu.VMEM((2, KG, K), jnp.int32),
            pltpu.VMEM((2, KG, K), jnp.int32),
            pltpu.VMEM((K, half), jnp.float32),
            pltpu.VMEM((K, half), jnp.float32),
            pltpu.VMEM((K, half), jnp.float32),
            pltpu.VMEM((K, half), jnp.float32),
            pltpu.VMEM((K, half), jnp.float32),
            pltpu.VMEM_SHARED((n, half), jnp.float32),
            *[pltpu.SemaphoreType.DMA for _ in range(8)],
        ],
    )
    def body(src_hbm, dst_hbm, y_hbm, e_hbm, r_hbm, out_hbm,
             src_v, dst_v, ge0, ge1, ge2, m0, m1, acc,
             se0, se1, se2, sg0, sg1, sg2, ss0, ss1):
        geb = (ge0, ge1, ge2)
        mb = (m0, m1)
        se = (se0, se1, se2)
        sg = (sg0, sg1, sg2)
        ss = (ss0, ss1)
        c = lax.axis_index("c")
        s = lax.axis_index("s")
        pltpu.sync_copy(r_hbm.at[pl.ds(c * n + s * rows, rows)],
                        acc.at[pl.ds(s * rows, rows)])
        if rem:
            @pl.when(s == 0)
            def _():
                pltpu.sync_copy(r_hbm.at[pl.ds(c * n + NS * rows, rem)],
                                acc.at[pl.ds(NS * rows, rem)])
        plsc.subcore_barrier()

        def stage_slabs(grp, gp):
            pltpu.sync_copy(src_hbm.at[c].at[s].at[pl.ds(grp * KG, KG)],
                            src_v.at[gp])
            pltpu.sync_copy(dst_hbm.at[s].at[pl.ds(grp * KG, KG)],
                            dst_v.at[gp])

        def issue_e(k, b3):
            base = c * e + s * eps + k * K
            pltpu.async_copy(e_hbm.at[pl.ds(base, K)], geb[b3], se[b3])

        def issue_gather(k, b3):
            grp = k // KG
            gp = grp % 2
            m = k % KG

            @pl.when(m == 0)
            def _():
                stage_slabs(grp, gp)
            pltpu.async_copy(y_hbm.at[src_v.at[gp].at[m]], geb[b3], sg[b3],
                             add=True)

        def drain(ref, sem):
            pltpu.make_async_copy(r_hbm.at[pl.ds(0, K)], ref, sem).wait()

        def compute(b3, b2):
            g_v, m_v = geb[b3], mb[b2]

            def row(i, carry2):
                for j in range(half // 16):
                    m_v[i, pl.ds(j * 16, 16)] = jnp.maximum(
                        g_v[i, pl.ds(j * 16, 16)], 0.0)
                return carry2

            lax.fori_loop(0, K, row, 0, unroll=8)

        def scatter(k, b2):
            grp = k // KG
            gp = grp % 2
            m = k % KG
            pltpu.async_copy(mb[b2], acc.at[dst_v.at[gp].at[m]], ss[b2],
                             add=True)

        def chunk(k, j3, j2):
            @pl.when(k + 1 < kchunks)
            def _():
                drain(geb[(j3 + 1) % 3], se[(j3 + 1) % 3])
                issue_gather(k + 1, (j3 + 1) % 3)
            drain(geb[j3], sg[j3])
            if isinstance(k, int) and k < 2:
                pass
            else:
                @pl.when(k >= 2)
                def _():
                    drain(mb[j2], ss[j2])
            compute(j3, j2)

            @pl.when(k + 3 < kchunks)
            def _():
                issue_e(k + 3, j3)
            scatter(k, j2)

        stage_slabs(0, 0)
        issue_e(0, 0)
        issue_e(1, 1)
        issue_e(2, 2)
        drain(geb[0], se[0])
        issue_gather(0, 0)

        sextets = kchunks // 6

        def sextet(t, carry):
            for j in range(6):
                chunk(6 * t + j, j % 3, j % 2)
            return carry

        lax.fori_loop(0, sextets, sextet, 0)
        for k in range(6 * sextets, kchunks):
            chunk(jnp.int32(k), k % 3, k % 2)
        drain(mb[0], ss[0])
        drain(mb[1], ss[1])
        plsc.subcore_barrier()
        pltpu.sync_copy(acc.at[pl.ds(s * rows, rows)],
                        out_hbm.at[pl.ds(c * n + s * rows, rows)])
        if rem:
            @pl.when(s == 0)
            def _():
                pltpu.sync_copy(acc.at[pl.ds(NS * rows, rem)],
                                out_hbm.at[pl.ds(c * n + NS * rows, rem)])

    return body


def _psi1_dense(x, edge_index, edge_attr, w1, w2, b2, wr):
    n, d = x.shape
    e = edge_index.shape[1]
    half = d // NC
    eps = e // NS
    kchunks = eps // K

    src = edge_index[0]
    dst = edge_index[1]
    kc_pad = (kchunks + KG - 1) // KG * KG
    src_pc = (src.reshape(1, NS, kchunks, K)
              + jnp.arange(NC, dtype=jnp.int32).reshape(NC, 1, 1, 1) * n)
    src_pc = jnp.pad(src_pc, ((0, 0), (0, 0), (0, kc_pad - kchunks), (0, 0)))
    dst3 = jnp.pad(dst.reshape(1, NS, kchunks, K),
                   ((0, 0), (0, 0), (0, kc_pad - kchunks), (0, 0)))[0]

    y2, r2 = _mm_node(x, w1, wr, 1000, half)
    e2 = _mm_edge(edge_attr, w2, b2, 2000, half)
    return src_pc, dst3, y2, e2, r2


def kernel(x_s, edge_index_s, edge_attr_s, batch_s,
           x_t, edge_index_t, edge_attr_t, batch_t,
           W_msg, b_msg, W_root):
    n, d = x_s.shape
    e = edge_index_s.shape[1]
    half = d // NC
    kchunks = (e // NS) // K
    w1 = W_msg[:d]
    w2 = W_msg[d:]
    b2 = b_msg.reshape(1, d)
    dense_s = _psi1_dense(x_s, edge_index_s, edge_attr_s, w1, w2, b2, W_root)
    dense_t = _psi1_dense(x_t, edge_index_t, edge_attr_t, w1, w2, b2, W_root)
    sc = _sc_edge_kernel(n, e, half, kchunks)
    agg_s = sc(*dense_s)
    agg_t = sc(*dense_t)
    h_s = _merge_relu(agg_s, n, 1000, half)
    h_t = _merge_relu(agg_t, n, 1000, half)
    return (h_s, h_t)


# speedup vs baseline: 1.0510x; 1.0510x over previous
import functools

import jax
import jax.numpy as jnp
import numpy as np
from jax import lax
from jax.experimental import pallas as pl
from jax.experimental.pallas import tpu as pltpu
from jax.experimental.pallas import tpu_sc as plsc

NC = 2
NS = 16
K = 40
KG = 8


def _mm_node_body(x_ref, w1_ref, wr_ref, y_ref, r_ref):
    x = x_ref[...]
    y_ref[...] = jnp.dot(x, w1_ref[...], preferred_element_type=jnp.float32)
    r_ref[...] = jnp.dot(x, wr_ref[...], preferred_element_type=jnp.float32)


def _mm_node(x, w1, wr, bn, half):
    n = x.shape[0]
    d = x.shape[1]
    nb = n // bn
    return pl.pallas_call(
        _mm_node_body,
        grid=(nb, NC),
        in_specs=[
            pl.BlockSpec((bn, d), lambda i, c: (i, 0)),
            pl.BlockSpec((d, half), lambda i, c: (0, c)),
            pl.BlockSpec((d, half), lambda i, c: (0, c)),
        ],
        out_specs=[
            pl.BlockSpec((bn, half), lambda i, c, _nb=nb: (c * _nb + i, 0)),
            pl.BlockSpec((bn, half), lambda i, c, _nb=nb: (c * _nb + i, 0)),
        ],
        out_shape=[
            jax.ShapeDtypeStruct((NC * n, half), jnp.float32),
            jax.ShapeDtypeStruct((NC * n, half), jnp.float32),
        ],
    )(x, w1, wr)


def _mm_edge_body(ea_ref, w2_ref, b_ref, e_ref):
    e_ref[...] = (jnp.dot(ea_ref[...], w2_ref[...],
                          preferred_element_type=jnp.float32)
                  + b_ref[...])


def _mm_edge(ea, w2, b2, be, half):
    e = ea.shape[0]
    de = ea.shape[1]
    eb = e // be
    return pl.pallas_call(
        _mm_edge_body,
        grid=(eb, NC),
        in_specs=[
            pl.BlockSpec((be, de), lambda i, c: (i, 0)),
            pl.BlockSpec((de, half), lambda i, c: (0, c)),
            pl.BlockSpec((1, half), lambda i, c: (0, c)),
        ],
        out_specs=pl.BlockSpec((be, half), lambda i, c, _eb=eb: (c * _eb + i, 0)),
        out_shape=jax.ShapeDtypeStruct((NC * e, half), jnp.float32),
    )(ea, w2, b2)


def _merge_body(a_ref, h_ref):
    h_ref[...] = jnp.maximum(a_ref[...], 0.0)


def _merge_relu(agg2, n, bn, half):
    nb = n // bn
    return pl.pallas_call(
        _merge_body,
        grid=(nb, NC),
        in_specs=[pl.BlockSpec((bn, half), lambda i, c, _nb=nb: (c * _nb + i, 0))],
        out_specs=pl.BlockSpec((bn, half), lambda i, c: (i, c)),
        out_shape=jax.ShapeDtypeStruct((n, NC * half), jnp.float32),
    )(agg2)


def _sc_edge_kernel(n, e, half, kchunks):
    eps = e // NS
    rows = (n // NS) // 16 * 16
    rem = n - NS * rows
    mesh = plsc.VectorSubcoreMesh(core_axis_name="c", subcore_axis_name="s")

    @functools.partial(
        pl.kernel,
        out_type=jax.ShapeDtypeStruct((NC * n, half), jnp.float32),
        mesh=mesh,
        scratch_types=[
            pltpu.VMEM((2, KG, K), jnp.int32),
            pltpu.VMEM((2, KG, K), jnp.int32),
            pltpu.VMEM((K, half), jnp.float32),
            pltpu.VMEM((K, half), jnp.float32),
            pltpu.VMEM((K, half), jnp.float32),
            pltpu.VMEM((K, half), jnp.float32),
            pltpu.VMEM((K, half), jnp.float32),
            pltpu.VMEM((K, half), jnp.float32),
            pltpu.VMEM_SHARED((n, half), jnp.float32),
            *[pltpu.SemaphoreType.DMA for _ in range(10)],
        ],
    )
    def body(src_hbm, dst_hbm, y_hbm, e_hbm, r_hbm, out_hbm,
             src_v, dst_v, ge0, ge1, ge2, ge3, m0, m1, acc,
             se0, se1, se2, se3, sg0, sg1, sg2, sg3, ss0, ss1):
        geb = (ge0, ge1, ge2, ge3)
        mb = (m0, m1)
        se = (se0, se1, se2, se3)
        sg = (sg0, sg1, sg2, sg3)
        ss = (ss0, ss1)
        c = lax.axis_index("c")
        s = lax.axis_index("s")
        pltpu.sync_copy(r_hbm.at[pl.ds(c * n + s * rows, rows)],
                        acc.at[pl.ds(s * rows, rows)])
        if rem:
            @pl.when(s == 0)
            def _():
                pltpu.sync_copy(r_hbm.at[pl.ds(c * n + NS * rows, rem)],
                                acc.at[pl.ds(NS * rows, rem)])
        plsc.subcore_barrier()

        def stage_slabs(grp, gp):
            pltpu.sync_copy(src_hbm.at[c].at[s].at[pl.ds(grp * KG, KG)],
                            src_v.at[gp])
            pltpu.sync_copy(dst_hbm.at[s].at[pl.ds(grp * KG, KG)],
                            dst_v.at[gp])

        def issue_e(k, b3):
            base = c * e + s * eps + k * K
            pltpu.async_copy(e_hbm.at[pl.ds(base, K)], geb[b3], se[b3])

        def issue_gather(k, b3):
            grp = k // KG
            gp = grp % 2
            m = k % KG

            @pl.when(m == 0)
            def _():
                stage_slabs(grp, gp)
            pltpu.async_copy(y_hbm.at[src_v.at[gp].at[m]], geb[b3], sg[b3],
                             add=True)

        def drain(ref, sem):
            pltpu.make_async_copy(r_hbm.at[pl.ds(0, K)], ref, sem).wait()

        def compute(b3, b2):
            g_v, m_v = geb[b3], mb[b2]

            def row(i, carry2):
                for j in range(half // 16):
                    m_v[i, pl.ds(j * 16, 16)] = jnp.maximum(
                        g_v[i, pl.ds(j * 16, 16)], 0.0)
                return carry2

            lax.fori_loop(0, K, row, 0, unroll=4)

        def scatter(k, b2):
            grp = k // KG
            gp = grp % 2
            m = k % KG
            pltpu.async_copy(mb[b2], acc.at[dst_v.at[gp].at[m]], ss[b2],
                             add=True)

        def chunk(k, j4, j2):
            @pl.when(k + 2 < kchunks)
            def _():
                drain(geb[(j4 + 2) % 4], se[(j4 + 2) % 4])
                issue_gather(k + 2, (j4 + 2) % 4)
            drain(geb[j4], sg[j4])

            @pl.when(k >= 2)
            def _():
                drain(mb[j2], ss[j2])
            compute(j4, j2)

            @pl.when(k + 4 < kchunks)
            def _():
                issue_e(k + 4, j4)
            scatter(k, j2)

        stage_slabs(0, 0)
        issue_e(0, 0)
        issue_e(1, 1)
        issue_e(2, 2)
        issue_e(3, 3)
        drain(geb[0], se[0])
        issue_gather(0, 0)
        drain(geb[1], se[1])
        issue_gather(1, 1)

        quartets = kchunks // 4

        def quartet(t, carry):
            for j in range(4):
                chunk(4 * t + j, j, j % 2)
            return carry

        lax.fori_loop(0, quartets, quartet, 0)
        for k in range(4 * quartets, kchunks):
            chunk(jnp.int32(k), k % 4, k % 2)
        drain(mb[0], ss[0])
        drain(mb[1], ss[1])
        plsc.subcore_barrier()
        pltpu.sync_copy(acc.at[pl.ds(s * rows, rows)],
                        out_hbm.at[pl.ds(c * n + s * rows, rows)])
        if rem:
            @pl.when(s == 0)
            def _():
                pltpu.sync_copy(acc.at[pl.ds(NS * rows, rem)],
                                out_hbm.at[pl.ds(c * n + NS * rows, rem)])

    return body


def _psi1_dense(x, edge_index, edge_attr, w1, w2, b2, wr):
    n, d = x.shape
    e = edge_index.shape[1]
    half = d // NC
    eps = e // NS
    kchunks = eps // K

    src = edge_index[0]
    dst = edge_index[1]
    kc_pad = (kchunks + KG - 1) // KG * KG
    src_pc = (src.reshape(1, NS, kchunks, K)
              + jnp.arange(NC, dtype=jnp.int32).reshape(NC, 1, 1, 1) * n)
    src_pc = jnp.pad(src_pc, ((0, 0), (0, 0), (0, kc_pad - kchunks), (0, 0)))
    dst3 = jnp.pad(dst.reshape(1, NS, kchunks, K),
                   ((0, 0), (0, 0), (0, kc_pad - kchunks), (0, 0)))[0]

    y2, r2 = _mm_node(x, w1, wr, 1000, half)
    e2 = _mm_edge(edge_attr, w2, b2, 2000, half)
    return src_pc, dst3, y2, e2, r2


def kernel(x_s, edge_index_s, edge_attr_s, batch_s,
           x_t, edge_index_t, edge_attr_t, batch_t,
           W_msg, b_msg, W_root):
    n, d = x_s.shape
    e = edge_index_s.shape[1]
    half = d // NC
    kchunks = (e // NS) // K
    w1 = W_msg[:d]
    w2 = W_msg[d:]
    b2 = b_msg.reshape(1, d)
    dense_s = _psi1_dense(x_s, edge_index_s, edge_attr_s, w1, w2, b2, W_root)
    dense_t = _psi1_dense(x_t, edge_index_t, edge_attr_t, w1, w2, b2, W_root)
    sc = _sc_edge_kernel(n, e, half, kchunks)
    agg_s = sc(*dense_s)
    agg_t = sc(*dense_t)
    h_s = _merge_relu(agg_s, n, 1000, half)
    h_t = _merge_relu(agg_t, n, 1000, half)
    return (h_s, h_t)
